# baseline (speedup 1.0000x reference)
import functools

import jax
import jax.numpy as jnp
from jax import lax
from jax.experimental import pallas as pl
from jax.experimental.pallas import tpu as pltpu
from jax.experimental.pallas import tpu_sc as plsc

_N = 10000
_D = 128
_E = 320000
_NC = 2
_NS = 16
_EPT = _E // (_NC * _NS)
_CHUNK = 112
_NCHUNK = 90
_EPTP = _CHUNK * _NCHUNK
_NACC = 10008
_PIECE = 80
_NPIECE = _N // _PIECE
_NBUF = 3
_NGRP = _NCHUNK // _NBUF


def _agg_body(x_hbm, idx_hbm, out_hbm, acc_sh, pk_b, rows_v, gsems, isems,
              ssems):
    c = lax.axis_index("c")
    s = lax.axis_index("s")
    w = c * _NS + s
    cbase = w * _NCHUNK
    npiece_mine = (_NPIECE + _NS - 1 - s) // _NS

    def idx_start(i, d):
        pltpu.async_copy(idx_hbm.at[cbase + i], pk_b[d], isems[d])

    def idx_wait(i, d):
        pltpu.make_async_copy(idx_hbm.at[cbase + i], pk_b[d], isems[d]).wait()

    def gather_start(b, d):
        pltpu.async_copy(x_hbm.at[pk_b[d].at[0]], rows_v[b], gsems[b])

    def gather_wait(b, d):
        pltpu.make_async_copy(x_hbm.at[pk_b[d].at[0]], rows_v[b],
                              gsems[b]).wait()

    def scatter_start(b, d):
        pltpu.async_copy(rows_v[b], acc_sh.at[pk_b[d].at[1]], ssems[b],
                         add=True)

    def scatter_wait(b, d):
        pltpu.make_async_copy(rows_v[b], acc_sh.at[pk_b[d].at[1]],
                              ssems[b]).wait()

    for i in range(_NBUF):
        idx_start(i, i)
    for i in range(_NBUF - 1):
        idx_wait(i, i)
        gather_start(i, i)

    stage = rows_v[2].at[pl.ds(0, _PIECE)]

    def _zstore(i, _):
        for j in range(_D // 16):
            rows_v[2][i, pl.ds(j * 16, 16)] = jnp.zeros((16,), jnp.float32)
        return 0

    lax.fori_loop(0, _PIECE, _zstore, 0)

    def _zpiece(i, _):
        row = (s + i * _NS) * _PIECE
        pltpu.sync_copy(stage, acc_sh.at[pl.ds(row, _PIECE)])
        return 0

    lax.fori_loop(0, npiece_mine, _zpiece, 0)
    plsc.subcore_barrier()

    def _stepk(i, k, first=False):
        b = k % _NBUF
        gather_wait(b, k)
        scatter_start(b, k)
        idx_start(i + 3, (k + 3) % 6)
        idx_wait(i + 2, (k + 2) % 6)
        if not first:
            scatter_wait((k + 2) % _NBUF, (k + 5) % 6)
        gather_start((k + 2) % _NBUF, (k + 2) % 6)

    for k in range(6):
        _stepk(k, k, first=(k == 0))

    def _sgroup(G, _):
        for k in range(6):
            _stepk(G * 6 + k, k)
        return 0

    _NSG = _NCHUNK // 6
    lax.fori_loop(1, _NSG - 1, _sgroup, 0)

    i0 = _NCHUNK - 6
    for k in range(3):
        _stepk(i0 + k, k)
    gather_wait(0, 3)
    scatter_start(0, 3)
    idx_wait(_NCHUNK - 1, 5)
    scatter_wait(2, 2)
    gather_start(2, 5)
    gather_wait(1, 4)
    scatter_start(1, 4)
    scatter_wait(0, 3)
    gather_wait(2, 5)
    scatter_start(2, 5)
    scatter_wait(1, 4)
    scatter_wait(2, 5)

    plsc.subcore_barrier()

    def _wpiece(i, _):
        row = (s + i * _NS) * _PIECE
        pltpu.async_copy(acc_sh.at[pl.ds(row, _PIECE)],
                         out_hbm.at[pl.ds(c * _N + row, _PIECE)], gsems[0])
        return 0

    lax.fori_loop(0, npiece_mine, _wpiece, 0)

    def _wdrain(i, _):
        row = (s + i * _NS) * _PIECE
        pltpu.make_async_copy(acc_sh.at[pl.ds(row, _PIECE)],
                              out_hbm.at[pl.ds(c * _N + row, _PIECE)],
                              gsems[0]).wait()
        return 0

    lax.fori_loop(0, npiece_mine, _wdrain, 0)


_agg = pl.kernel(
    _agg_body,
    out_type=jax.ShapeDtypeStruct((_NC * _N, _D), jnp.float32),
    mesh=plsc.VectorSubcoreMesh(core_axis_name="c", subcore_axis_name="s"),
    scratch_types=[
        pltpu.VMEM_SHARED((_NACC, _D), jnp.float32),
        [pltpu.VMEM((2, _CHUNK), jnp.int32) for _ in range(2 * _NBUF)],
        [pltpu.VMEM((_CHUNK, _D), jnp.float32) for _ in range(_NBUF)],
        [pltpu.SemaphoreType.DMA for _ in range(_NBUF)],
        [pltpu.SemaphoreType.DMA for _ in range(2 * _NBUF)],
        [pltpu.SemaphoreType.DMA for _ in range(_NBUF)],
    ],
)


def _mm_body(relu, p0_ref, p1_ref, x_ref, wrelT_ref, wrootT_ref, b_ref, o_ref):
    agg = p0_ref[...] + p1_ref[...]
    out = jnp.dot(agg, wrelT_ref[...],
                  preferred_element_type=jnp.float32,
                  precision=lax.Precision.HIGHEST)
    out = out + jnp.dot(x_ref[...], wrootT_ref[...],
                        preferred_element_type=jnp.float32,
                        precision=lax.Precision.HIGHEST)
    out = out + b_ref[...]
    if relu:
        out = jnp.maximum(out, 0.0)
    o_ref[...] = out


def _mm(p, x, wrelT, wrootT, b2d, relu):
    blk = 5000
    return pl.pallas_call(
        functools.partial(_mm_body, relu),
        grid=(_N // blk,),
        in_specs=[
            pl.BlockSpec((blk, _D), lambda i: (i, 0)),
            pl.BlockSpec((blk, _D), lambda i: (i + _N // 5000, 0)),
            pl.BlockSpec((blk, _D), lambda i: (i, 0)),
            pl.BlockSpec((_D, _D), lambda i: (0, 0)),
            pl.BlockSpec((_D, _D), lambda i: (0, 0)),
            pl.BlockSpec((1, _D), lambda i: (0, 0)),
        ],
        out_specs=pl.BlockSpec((blk, _D), lambda i: (i, 0)),
        out_shape=jax.ShapeDtypeStruct((_N, _D), jnp.float32),
    )(p, p, x, wrelT, wrootT, b2d)


def kernel(x, edge_index, W1_rel, W1_root, b1, W2_rel, W2_root, b2):
    nw = _NC * _NS
    pad = _EPTP - _EPT
    src = jnp.concatenate(
        [edge_index[0].reshape(nw, _EPT),
         jnp.zeros((nw, pad), jnp.int32)], axis=1)
    dst = jnp.concatenate(
        [edge_index[1].reshape(nw, _EPT),
         jnp.full((nw, pad), _N, jnp.int32)], axis=1)
    idx = jnp.stack(
        [src.reshape(nw, _NCHUNK, _CHUNK), dst.reshape(nw, _NCHUNK, _CHUNK)],
        axis=2).reshape(nw * _NCHUNK, 2, _CHUNK)
    p = _agg(x, idx)
    h = _mm(p, x, W1_rel.T, W1_root.T, b1.reshape(1, _D), True)
    p = _agg(h, idx)
    return _mm(p, h, W2_rel.T, W2_root.T, b2.reshape(1, _D), False)


# speedup vs baseline: 1.0373x; 1.0373x over previous
import functools

import jax
import jax.numpy as jnp
from jax import lax
from jax.experimental import pallas as pl
from jax.experimental.pallas import tpu as pltpu
from jax.experimental.pallas import tpu_sc as plsc

_N = 10000
_D = 128
_E = 320000
_NC = 2
_NS = 16
_EPT = _E // (_NC * _NS)
_CHUNK = 112
_NCHUNK = 90
_EPTP = _CHUNK * _NCHUNK
_NACC = 10008
_PIECE = 80
_NPIECE = _N // _PIECE
_NBUF = 3
_NGRP = _NCHUNK // _NBUF


def _agg_body(x_hbm, idx_hbm, out_hbm, acc_sh, pk_b, rows_v, gsems, isems,
              ssems):
    c = lax.axis_index("c")
    s = lax.axis_index("s")
    w = c * _NS + s
    cbase = w * _NCHUNK
    npiece_mine = (_NPIECE + _NS - 1 - s) // _NS

    def idx_start(i, d):
        pltpu.async_copy(idx_hbm.at[cbase + i], pk_b[d], isems[d])

    def idx_wait(i, d):
        pltpu.make_async_copy(idx_hbm.at[cbase + i], pk_b[d], isems[d]).wait()

    def gather_start(b, d):
        pltpu.async_copy(x_hbm.at[pk_b[d].at[0]], rows_v[b], gsems[b])

    def gather_wait(b, d):
        pltpu.make_async_copy(x_hbm.at[pk_b[d].at[0]], rows_v[b],
                              gsems[b]).wait()

    def scatter_start(b, d):
        pltpu.async_copy(rows_v[b], acc_sh.at[pk_b[d].at[1]], ssems[b],
                         add=True)

    def scatter_wait(b, d):
        pltpu.make_async_copy(rows_v[b], acc_sh.at[pk_b[d].at[1]],
                              ssems[b]).wait()

    for i in range(_NBUF):
        idx_start(i, i)
    for i in range(_NBUF - 1):
        idx_wait(i, i)
        gather_start(i, i)

    stage = rows_v[2].at[pl.ds(0, _PIECE)]

    def _zstore(i, _):
        for j in range(_D // 16):
            rows_v[2][i, pl.ds(j * 16, 16)] = jnp.zeros((16,), jnp.float32)
        return 0

    lax.fori_loop(0, _PIECE, _zstore, 0)

    def _zpiece(i, _):
        row = (s + i * _NS) * _PIECE
        pltpu.sync_copy(stage, acc_sh.at[pl.ds(row, _PIECE)])
        return 0

    lax.fori_loop(0, npiece_mine, _zpiece, 0)
    plsc.subcore_barrier()

    def _stepk(i, k, first=False):
        b = k % _NBUF
        gather_wait(b, k)
        scatter_start(b, k)
        idx_start(i + 3, (k + 3) % 6)
        idx_wait(i + 2, (k + 2) % 6)
        if not first:
            scatter_wait((k + 2) % _NBUF, (k + 5) % 6)
        gather_start((k + 2) % _NBUF, (k + 2) % 6)

    for k in range(6):
        _stepk(k, k, first=(k == 0))

    def _sgroup(G, _):
        for k in range(6):
            _stepk(G * 6 + k, k)
        return 0

    _NSG = _NCHUNK // 6
    lax.fori_loop(1, _NSG - 1, _sgroup, 0)

    i0 = _NCHUNK - 6
    for k in range(3):
        _stepk(i0 + k, k)
    gather_wait(0, 3)
    scatter_start(0, 3)
    idx_wait(_NCHUNK - 1, 5)
    scatter_wait(2, 2)
    gather_start(2, 5)
    gather_wait(1, 4)
    scatter_start(1, 4)
    scatter_wait(0, 3)
    gather_wait(2, 5)
    scatter_start(2, 5)
    scatter_wait(1, 4)
    scatter_wait(2, 5)

    plsc.subcore_barrier()

    def _wpiece(i, _):
        row = (s + i * _NS) * _PIECE
        pltpu.async_copy(acc_sh.at[pl.ds(row, _PIECE)],
                         out_hbm.at[pl.ds(c * _N + row, _PIECE)], gsems[0])
        return 0

    lax.fori_loop(0, npiece_mine, _wpiece, 0)

    def _wdrain(i, _):
        row = (s + i * _NS) * _PIECE
        pltpu.make_async_copy(acc_sh.at[pl.ds(row, _PIECE)],
                              out_hbm.at[pl.ds(c * _N + row, _PIECE)],
                              gsems[0]).wait()
        return 0

    lax.fori_loop(0, npiece_mine, _wdrain, 0)


_agg = pl.kernel(
    _agg_body,
    out_type=jax.ShapeDtypeStruct((_NC * _N, _D), jnp.float32),
    mesh=plsc.VectorSubcoreMesh(core_axis_name="c", subcore_axis_name="s"),
    scratch_types=[
        pltpu.VMEM_SHARED((_NACC, _D), jnp.float32),
        [pltpu.VMEM((2, _CHUNK), jnp.int32) for _ in range(2 * _NBUF)],
        [pltpu.VMEM((_CHUNK, _D), jnp.float32) for _ in range(_NBUF)],
        [pltpu.SemaphoreType.DMA for _ in range(_NBUF)],
        [pltpu.SemaphoreType.DMA for _ in range(2 * _NBUF)],
        [pltpu.SemaphoreType.DMA for _ in range(_NBUF)],
    ],
)


def _mm_body(relu, p0_ref, p1_ref, x_ref, wrelT_ref, wrootT_ref, b_ref, o_ref):
    agg = p0_ref[...] + p1_ref[...]
    out = jnp.dot(agg, wrelT_ref[...],
                  preferred_element_type=jnp.float32,
                  precision=lax.Precision.HIGHEST)
    out = out + jnp.dot(x_ref[...], wrootT_ref[...],
                        preferred_element_type=jnp.float32,
                        precision=lax.Precision.HIGHEST)
    out = out + b_ref[...]
    if relu:
        out = jnp.maximum(out, 0.0)
    o_ref[...] = out


def _mm(p, x, wrelT, wrootT, b2d, relu):
    blk = 2000
    return pl.pallas_call(
        functools.partial(_mm_body, relu),
        grid=(_N // blk,),
        in_specs=[
            pl.BlockSpec((blk, _D), lambda i: (i, 0)),
            pl.BlockSpec((blk, _D), lambda i: (i + _N // 2000, 0)),
            pl.BlockSpec((blk, _D), lambda i: (i, 0)),
            pl.BlockSpec((_D, _D), lambda i: (0, 0)),
            pl.BlockSpec((_D, _D), lambda i: (0, 0)),
            pl.BlockSpec((1, _D), lambda i: (0, 0)),
        ],
        out_specs=pl.BlockSpec((blk, _D), lambda i: (i, 0)),
        out_shape=jax.ShapeDtypeStruct((_N, _D), jnp.float32),
    )(p, p, x, wrelT, wrootT, b2d)


def kernel(x, edge_index, W1_rel, W1_root, b1, W2_rel, W2_root, b2):
    nw = _NC * _NS
    pad = _EPTP - _EPT
    src = jnp.concatenate(
        [edge_index[0].reshape(nw, _EPT),
         jnp.zeros((nw, pad), jnp.int32)], axis=1)
    dst = jnp.concatenate(
        [edge_index[1].reshape(nw, _EPT),
         jnp.full((nw, pad), _N, jnp.int32)], axis=1)
    idx = jnp.stack(
        [src.reshape(nw, _NCHUNK, _CHUNK), dst.reshape(nw, _NCHUNK, _CHUNK)],
        axis=2).reshape(nw * _NCHUNK, 2, _CHUNK)
    p = _agg(x, idx)
    h = _mm(p, x, W1_rel.T, W1_root.T, b1.reshape(1, _D), True)
    p = _agg(h, idx)
    return _mm(p, h, W2_rel.T, W2_root.T, b2.reshape(1, _D), False)
